# baseline (speedup 1.0000x reference)
import jax
import jax.numpy as jnp
from jax.experimental import pallas as pl
from jax.experimental.pallas import tpu as pltpu

B = 4096
X_DIM = 784
N_CLASSES = 10
HIDDEN = 2048
EMBED_DIM = 1024
LATENT_DIM = 64
K_CODES = 1024

TILE = 512

_DN_T = (((1,), (1,)), ((), ()))


def _fwd_kernel(x_ref, c_ref, we1_ref, be1_ref, we2_ref, be2_ref,
                wfc_ref, bfc_ref, cb_ref, wd1_ref, bd1_ref, wd2_ref,
                bd2_ref, recon_ref, ze_ref, zq_ref,
                s_e1, s_e2, s_d1, s_d2, s_cbhi, s_cblo, s_cbT):
    f32 = jnp.float32
    bf16 = jnp.bfloat16

    @pl.when(pl.program_id(1) == 0)
    def _prep():
        s_e1[...] = we1_ref[...].astype(bf16)
        s_e2[...] = we2_ref[...].astype(bf16)
        s_d1[...] = wd1_ref[...].astype(bf16)
        s_d2[...] = wd2_ref[...].astype(bf16)
        cb = cb_ref[...]
        cb_hi = cb.astype(bf16)
        s_cbhi[...] = cb_hi
        s_cblo[...] = (cb - cb_hi.astype(f32)).astype(bf16)
        s_cbT[...] = cb.T

    x = x_ref[...].astype(bf16)
    i = pl.program_id(0) * pl.num_programs(1) + pl.program_id(1)
    cmat = c_ref[pl.ds(i * TILE, TILE)].reshape(TILE, 1)
    cls_iota = jax.lax.broadcasted_iota(jnp.int32, (TILE, N_CLASSES), 1)
    oh = (cmat == cls_iota).astype(bf16)

    xcat = jnp.concatenate([x, oh], axis=1)
    h = jnp.dot(xcat, s_e1[...], preferred_element_type=f32)
    h = jnp.maximum(h + be1_ref[...], 0.0)
    enc = jnp.maximum(
        jnp.dot(h.astype(bf16), s_e2[...], preferred_element_type=f32)
        + be2_ref[...], 0.0)
    z_e = jnp.dot(enc, wfc_ref[...], preferred_element_type=f32) + bfc_ref[...]

    cbT = s_cbT[...]
    ze2 = jnp.sum(z_e * z_e, axis=-1, keepdims=True)
    cb2 = jnp.sum(cbT * cbT, axis=0, keepdims=True)
    cross = jnp.dot(z_e, cbT, preferred_element_type=f32)
    d2 = ze2 - 2.0 * cross + cb2
    rowmin = jnp.min(d2, axis=-1, keepdims=True)
    code_iota = jax.lax.broadcasted_iota(jnp.int32, (TILE, K_CODES), 1)
    cand = jnp.where(d2 == rowmin, code_iota, K_CODES)
    idx = jnp.min(cand, axis=-1, keepdims=True)
    qoh = (code_iota == idx).astype(bf16)
    quant = (jnp.dot(qoh, s_cbhi[...], preferred_element_type=f32)
             + jnp.dot(qoh, s_cblo[...], preferred_element_type=f32))
    z_q = z_e + (quant - z_e)

    zcat = jnp.concatenate([z_q.astype(bf16), oh], axis=1)
    hd = jnp.dot(zcat, s_d1[...], preferred_element_type=f32)
    hd = jnp.maximum(hd + bd1_ref[...], 0.0)
    recon = jax.nn.sigmoid(
        jnp.dot(hd.astype(bf16), s_d2[...], preferred_element_type=f32)
        + bd2_ref[...])

    recon_ref[...] = recon
    ze_ref[...] = z_e
    zq_ref[...] = z_q


def kernel(x, c, W_e1, b_e1, W_e2, b_e2, W_fc, b_fc, codebook,
           W_d1, b_d1, W_d2, b_d2):
    n_tiles = B // TILE
    bf16 = jnp.bfloat16
    args = (
        x,
        c.astype(jnp.int32),
        W_e1,
        b_e1.reshape(1, HIDDEN),
        W_e2,
        b_e2.reshape(1, EMBED_DIM),
        W_fc,
        b_fc.reshape(1, LATENT_DIM),
        codebook,
        W_d1,
        b_d1.reshape(1, HIDDEN),
        W_d2,
        b_d2.reshape(1, X_DIM),
    )

    n_inner = n_tiles // 2

    def tiled(ncols):
        return pl.BlockSpec((TILE, ncols), lambda i, j: (i * n_inner + j, 0))

    def whole(a):
        return pl.BlockSpec(a.shape, lambda i, j: tuple(0 for _ in a.shape))

    in_specs = [
        tiled(X_DIM),
    ] + [whole(a) for a in args[1:]]

    out_shape = (
        jax.ShapeDtypeStruct((B, X_DIM), jnp.float32),
        jax.ShapeDtypeStruct((B, LATENT_DIM), jnp.float32),
        jax.ShapeDtypeStruct((B, LATENT_DIM), jnp.float32),
    )
    out_specs = (
        tiled(X_DIM),
        tiled(LATENT_DIM),
        tiled(LATENT_DIM),
    )

    scratch_shapes = [
        pltpu.VMEM((X_DIM + N_CLASSES, HIDDEN), bf16),
        pltpu.VMEM((HIDDEN, EMBED_DIM), bf16),
        pltpu.VMEM((LATENT_DIM + N_CLASSES, HIDDEN), bf16),
        pltpu.VMEM((HIDDEN, X_DIM), bf16),
        pltpu.VMEM((K_CODES, LATENT_DIM), bf16),
        pltpu.VMEM((K_CODES, LATENT_DIM), bf16),
        pltpu.VMEM((LATENT_DIM, K_CODES), jnp.float32),
    ]

    recon, z_e, z_q = pl.pallas_call(
        _fwd_kernel,
        grid=(2, n_inner),
        in_specs=in_specs,
        out_specs=out_specs,
        out_shape=out_shape,
        scratch_shapes=scratch_shapes,
        compiler_params=pltpu.CompilerParams(
            dimension_semantics=("parallel", "arbitrary")),
    )(*args)
    return (recon, z_e, z_q)
